# Initial kernel scaffold; baseline (speedup 1.0000x reference)
import jax
import jax.numpy as jnp
from jax.experimental import pallas as pl


def kernel(x, edge_index, batch, Wp, bp, Wc, bc, lng, lnb, Wg1, bg1, gg, gb, Wg2, bg2, Wg3, bg3, Wn1, bn1, ng, nb, Wn2, bn2, Wn3, bn3):
    raise NotImplementedError("write your pallas kernel here")


# speedup vs baseline: 4.5624x; 4.5624x over previous
import functools

import jax
import jax.numpy as jnp
from jax import lax
from jax.experimental import pallas as pl
from jax.experimental.pallas import tpu as pltpu
from jax.experimental.pallas import tpu_sc as plsc

N = 10000
D = 128
G = 16
EPS = 1e-5

NPAD = 10240
BR = 512
NB = NPAD // BR
NC = 2
NS = 16
NW = NC * NS
LANES = 128
RPS = NPAD // NS

_HI = None


def _mesh():
    return plsc.VectorSubcoreMesh(core_axis_name="c", subcore_axis_name="s")


def _sc_degree(sd2, z128, ones128):
    cpw = sd2.shape[1]

    @functools.partial(
        pl.kernel,
        out_type=jax.ShapeDtypeStruct((NC * NPAD, D), jnp.float32),
        mesh=_mesh(),
        scratch_types=[
            pltpu.VMEM((4, 2, LANES), jnp.int32),
            pltpu.VMEM((LANES, D), jnp.float32),
            pltpu.VMEM_SHARED((NPAD, D), jnp.float32),
            pltpu.SemaphoreType.DMA,
            pltpu.SemaphoreType.DMA,
            pltpu.SemaphoreType.DMA,
            pltpu.SemaphoreType.DMA,
        ],
    )
    def deg_kernel(sd_hbm, z_hbm, ones_hbm, out_hbm, idxv, onesv, acc,
                   si0, si1, si2, si3):
        cid = lax.axis_index("c")
        sid = lax.axis_index("s")
        wid = sid * NC + cid
        isems = (si0, si1, si2, si3)
        pltpu.sync_copy(ones_hbm, onesv)
        pltpu.sync_copy(z_hbm.at[pl.ds(sid * RPS, RPS)],
                        acc.at[pl.ds(sid * RPS, RPS)])
        for r in range(4):
            pltpu.async_copy(sd_hbm.at[wid, r], idxv.at[r], isems[r])
        plsc.subcore_barrier()

        def body(i, carry):
            for b in range(4):
                c = i * 4 + b
                pltpu.make_async_copy(sd_hbm.at[0, 0], idxv.at[b],
                                      isems[b]).wait()
                pltpu.sync_copy(onesv, acc.at[idxv.at[b, 1]], add=True)

                @pl.when(c + 4 < cpw)
                def _next_idx():
                    pltpu.async_copy(sd_hbm.at[wid, c + 4], idxv.at[b],
                                     isems[b])
            return carry

        lax.fori_loop(0, cpw // 4, body, 0)
        plsc.subcore_barrier()
        pltpu.sync_copy(acc.at[pl.ds(sid * RPS, RPS)],
                        out_hbm.at[pl.ds(cid * NPAD + sid * RPS, RPS)])

    return deg_kernel(sd2, z128, ones128)


def _sc_scatter(hs, sd2, z128):
    cpw = sd2.shape[1]

    @functools.partial(
        pl.kernel,
        out_type=jax.ShapeDtypeStruct((NC * NPAD, D), jnp.float32),
        mesh=_mesh(),
        scratch_types=[
            pltpu.VMEM((4, 2, LANES), jnp.int32),
            pltpu.VMEM((2, LANES, D), jnp.float32),
            pltpu.VMEM_SHARED((NPAD, D), jnp.float32),
            pltpu.SemaphoreType.DMA,
            pltpu.SemaphoreType.DMA,
            pltpu.SemaphoreType.DMA,
            pltpu.SemaphoreType.DMA,
            pltpu.SemaphoreType.DMA,
            pltpu.SemaphoreType.DMA,
        ],
    )
    def mp_kernel(hs_hbm, sd_hbm, z_hbm, out_hbm, idxv, rows, acc,
                  si0, si1, si2, si3, sg0, sg1):
        cid = lax.axis_index("c")
        sid = lax.axis_index("s")
        wid = sid * NC + cid
        isems = (si0, si1, si2, si3)
        gsems = (sg0, sg1)
        pltpu.sync_copy(z_hbm.at[pl.ds(sid * RPS, RPS)],
                        acc.at[pl.ds(sid * RPS, RPS)])

        def idx_start(c, ring):
            return pltpu.async_copy(sd_hbm.at[wid, c], idxv.at[ring],
                                    isems[ring])

        def idx_wait(ring):
            pltpu.make_async_copy(sd_hbm.at[0, 0], idxv.at[ring],
                                  isems[ring]).wait()

        def gather_start(c, ring, rb):
            return pltpu.async_copy(hs_hbm.at[idxv.at[ring, 0]],
                                    rows.at[rb], gsems[rb])

        def gather_wait(ring, rb):
            pltpu.make_async_copy(hs_hbm.at[idxv.at[ring, 0]], rows.at[rb],
                                  gsems[rb]).wait()

        for r in range(4):
            idx_start(r, r)
        plsc.subcore_barrier()
        idx_wait(0)
        gather_start(0, 0, 0)

        def body(i, carry):
            for b in range(4):
                c = i * 4 + b
                rb = b % 2
                gather_wait(b, rb)

                @pl.when(c + 1 < cpw)
                def _next_gather():
                    idx_wait((b + 1) % 4)
                    gather_start(c + 1, (b + 1) % 4, 1 - rb)

                pltpu.sync_copy(rows.at[rb], acc.at[idxv.at[b, 1]], add=True)

                @pl.when(c + 4 < cpw)
                def _next_idx():
                    idx_start(c + 4, b)
            return carry

        lax.fori_loop(0, cpw // 4, body, 0)
        plsc.subcore_barrier()
        pltpu.sync_copy(acc.at[pl.ds(sid * RPS, RPS)],
                        out_hbm.at[pl.ds(cid * NPAD + sid * RPS, RPS)])

    return mp_kernel(hs, sd2, z128)


def _dinv_col(d0, d1):
    d = d0[:, :1] + d1[:, :1]
    return jnp.where(d > 0, 1.0 / jnp.sqrt(jnp.maximum(d, 1.0)), 0.0)


def _tc_prep(xp, Wp, bp, Wc0, deg0, deg1):
    def body(x_ref, wp_ref, bp_ref, wc_ref, d0_ref, d1_ref, h_ref, hs_ref):
        h = jnp.maximum(
            jnp.dot(x_ref[...], wp_ref[...], precision=_HI) + bp_ref[...], 0.0)
        dinv = _dinv_col(d0_ref[...], d1_ref[...])
        h_ref[...] = h
        hs_ref[...] = jnp.dot(h, wc_ref[...], precision=_HI) * dinv

    return pl.pallas_call(
        body,
        grid=(NB,),
        in_specs=[
            pl.BlockSpec((BR, D), lambda i: (i, 0)),
            pl.BlockSpec((D, D), lambda i: (0, 0)),
            pl.BlockSpec((1, D), lambda i: (0, 0)),
            pl.BlockSpec((D, D), lambda i: (0, 0)),
            pl.BlockSpec((BR, D), lambda i: (i, 0)),
            pl.BlockSpec((BR, D), lambda i: (i, 0)),
        ],
        out_specs=[pl.BlockSpec((BR, D), lambda i: (i, 0)),
                   pl.BlockSpec((BR, D), lambda i: (i, 0))],
        out_shape=[jax.ShapeDtypeStruct((NPAD, D), jnp.float32),
                   jax.ShapeDtypeStruct((NPAD, D), jnp.float32)],
    )(xp, Wp, bp, Wc0, deg0, deg1)


def _tc_layer(a0, a1, deg0, deg1, hprev, bci, gi, bi, Wnext):
    has_next = Wnext is not None

    def body(a0_ref, a1_ref, d0_ref, d1_ref, h_ref, bc_ref, g_ref, b_ref,
             *rest):
        if has_next:
            w_ref, oh_ref, ohs_ref = rest
        else:
            (oh_ref,) = rest
        dinv = _dinv_col(d0_ref[...], d1_ref[...])
        agg = (a0_ref[...] + a1_ref[...]) * dinv + bc_ref[...]
        m = jnp.mean(agg, axis=-1, keepdims=True)
        v = jnp.mean((agg - m) ** 2, axis=-1, keepdims=True)
        hn = (agg - m) / jnp.sqrt(v + EPS) * g_ref[...] + b_ref[...]
        h = jnp.maximum(hn, 0.0) + h_ref[...]
        oh_ref[...] = h
        if has_next:
            ohs_ref[...] = jnp.dot(h, w_ref[...], precision=_HI) * dinv

    in_specs = [
        pl.BlockSpec((BR, D), lambda i: (i, 0)),
        pl.BlockSpec((BR, D), lambda i: (i, 0)),
        pl.BlockSpec((BR, D), lambda i: (i, 0)),
        pl.BlockSpec((BR, D), lambda i: (i, 0)),
        pl.BlockSpec((BR, D), lambda i: (i, 0)),
        pl.BlockSpec((1, D), lambda i: (0, 0)),
        pl.BlockSpec((1, D), lambda i: (0, 0)),
        pl.BlockSpec((1, D), lambda i: (0, 0)),
    ]
    args = [a0, a1, deg0, deg1, hprev, bci, gi, bi]
    out_specs = [pl.BlockSpec((BR, D), lambda i: (i, 0))]
    out_shape = [jax.ShapeDtypeStruct((NPAD, D), jnp.float32)]
    if has_next:
        in_specs.append(pl.BlockSpec((D, D), lambda i: (0, 0)))
        args.append(Wnext)
        out_specs.append(pl.BlockSpec((BR, D), lambda i: (i, 0)))
        out_shape.append(jax.ShapeDtypeStruct((NPAD, D), jnp.float32))
    return pl.pallas_call(
        body, grid=(NB,), in_specs=in_specs,
        out_specs=out_specs, out_shape=out_shape)(*args)


def _tc_node_head(h, btf, Wn1, bn1, ng, nb_, Wn2, bn2, Wn3p, bn3p):
    def body(h_ref, bt_ref, w1, b1, g1, bb1, w2, b2, w3, b3,
             nl_ref, gs_ref, gm_ref, gc_ref):
        i = pl.program_id(0)
        hb = h_ref[...]
        n1 = jnp.dot(hb, w1[...], precision=_HI) + b1[...]
        m = jnp.mean(n1, axis=-1, keepdims=True)
        v = jnp.mean((n1 - m) ** 2, axis=-1, keepdims=True)
        n1 = jnp.maximum((n1 - m) / jnp.sqrt(v + EPS) * g1[...] + bb1[...],
                         0.0)
        n2 = jnp.maximum(jnp.dot(n1, w2[...], precision=_HI) + b2[...], 0.0)
        nl_ref[...] = jnp.dot(n2, w3[...], precision=_HI) + b3[...]

        @pl.when(i == 0)
        def _init():
            gs_ref[...] = jnp.zeros_like(gs_ref)
            gm_ref[...] = jnp.full_like(gm_ref, -jnp.inf)
            gc_ref[...] = jnp.zeros_like(gc_ref)

        bt = bt_ref[:, :1]
        for g in range(G):
            mask = bt == float(g)

            @pl.when(jnp.any(mask))
            def _upd():
                gs_ref[g:g + 1, :] += jnp.sum(
                    jnp.where(mask, hb, 0.0), axis=0, keepdims=True)
                gm_ref[g:g + 1, :] = jnp.maximum(
                    gm_ref[g:g + 1, :],
                    jnp.max(jnp.where(mask, hb, -jnp.inf), axis=0,
                            keepdims=True))
                gc_ref[g:g + 1, :] += jnp.sum(mask.astype(jnp.float32))

    return pl.pallas_call(
        body,
        grid=(NB,),
        in_specs=[
            pl.BlockSpec((BR, D), lambda i: (i, 0)),
            pl.BlockSpec((BR, 8), lambda i: (i, 0)),
            pl.BlockSpec((D, D), lambda i: (0, 0)),
            pl.BlockSpec((1, D), lambda i: (0, 0)),
            pl.BlockSpec((1, D), lambda i: (0, 0)),
            pl.BlockSpec((1, D), lambda i: (0, 0)),
            pl.BlockSpec((D, D // 2), lambda i: (0, 0)),
            pl.BlockSpec((1, D // 2), lambda i: (0, 0)),
            pl.BlockSpec((D // 2, 8), lambda i: (0, 0)),
            pl.BlockSpec((1, 8), lambda i: (0, 0)),
        ],
        out_specs=[
            pl.BlockSpec((BR, 8), lambda i: (i, 0)),
            pl.BlockSpec((G, D), lambda i: (0, 0)),
            pl.BlockSpec((G, D), lambda i: (0, 0)),
            pl.BlockSpec((G, D), lambda i: (0, 0)),
        ],
        out_shape=[
            jax.ShapeDtypeStruct((NPAD, 8), jnp.float32),
            jax.ShapeDtypeStruct((G, D), jnp.float32),
            jax.ShapeDtypeStruct((G, D), jnp.float32),
            jax.ShapeDtypeStruct((G, D), jnp.float32),
        ],
    )(h, btf, Wn1, bn1, ng, nb_, Wn2, bn2, Wn3p, bn3p)


def _tc_graph_head(gs, gm, gc, W1a, W1b, b1, g1, bb1, W2, b2, W3p, b3p):
    def body(gs_ref, gm_ref, gc_ref, w1a, w1b, bb, gg_, gbb, w2, b2_, w3, b3_,
             out_ref):
        cnt = jnp.maximum(gc_ref[...][:, :1], 1.0)
        gmean = gs_ref[...] / cnt
        e = (jnp.dot(gmean, w1a[...], precision=_HI)
             + jnp.dot(gm_ref[...], w1b[...], precision=_HI) + bb[...])
        m = jnp.mean(e, axis=-1, keepdims=True)
        v = jnp.mean((e - m) ** 2, axis=-1, keepdims=True)
        h1 = jnp.maximum((e - m) / jnp.sqrt(v + EPS) * gg_[...] + gbb[...],
                         0.0)
        h2 = jnp.maximum(jnp.dot(h1, w2[...], precision=_HI) + b2_[...], 0.0)
        out_ref[...] = jnp.dot(h2, w3[...], precision=_HI) + b3_[...]

    return pl.pallas_call(
        body,
        grid=(1,),
        in_specs=[
            pl.BlockSpec((G, D), lambda i: (0, 0)),
            pl.BlockSpec((G, D), lambda i: (0, 0)),
            pl.BlockSpec((G, D), lambda i: (0, 0)),
            pl.BlockSpec((D, D), lambda i: (0, 0)),
            pl.BlockSpec((D, D), lambda i: (0, 0)),
            pl.BlockSpec((1, D), lambda i: (0, 0)),
            pl.BlockSpec((1, D), lambda i: (0, 0)),
            pl.BlockSpec((1, D), lambda i: (0, 0)),
            pl.BlockSpec((D, D // 2), lambda i: (0, 0)),
            pl.BlockSpec((1, D // 2), lambda i: (0, 0)),
            pl.BlockSpec((D // 2, 8), lambda i: (0, 0)),
            pl.BlockSpec((1, 8), lambda i: (0, 0)),
        ],
        out_specs=pl.BlockSpec((G, 8), lambda i: (0, 0)),
        out_shape=jax.ShapeDtypeStruct((G, 8), jnp.float32),
    )(gs, gm, gc, W1a, W1b, b1, g1, bb1, W2, b2, W3p, b3p)


def kernel(x, edge_index, batch, Wp, bp, Wc, bc, lng, lnb, Wg1, bg1, gg, gb,
           Wg2, bg2, Wg3, bg3, Wn1, bn1, ng, nb, Wn2, bn2, Wn3, bn3):
    ei = edge_index.astype(jnp.int32)
    e_cnt = ei.shape[1]
    loops = jnp.arange(N, dtype=jnp.int32)
    src = jnp.concatenate([ei[0], loops])
    dst = jnp.concatenate([ei[1], loops])
    et = e_cnt + N
    cpw = -(-et // (NW * LANES))
    cpw = -(-cpw // 4) * 4
    ep = NW * cpw * LANES
    pad = jnp.full((ep - et,), N, jnp.int32)
    src2 = jnp.concatenate([src, pad]).reshape(NW, cpw, LANES)
    dst2 = jnp.concatenate([dst, pad]).reshape(NW, cpw, LANES)
    sd2 = jnp.stack([src2, dst2], axis=2)

    z128 = jnp.zeros((NPAD, D), jnp.float32)
    ones128 = jnp.ones((LANES, D), jnp.float32)

    deg2 = _sc_degree(sd2, z128, ones128)
    deg0, deg1 = deg2[:NPAD], deg2[NPAD:]

    xp = jnp.zeros((NPAD, D), jnp.float32).at[:N].set(x)
    h, hs = _tc_prep(xp, Wp, bp.reshape(1, D), Wc[0], deg0, deg1)
    for i in range(4):
        acc2 = _sc_scatter(hs, sd2, z128)
        a0, a1 = acc2[:NPAD], acc2[NPAD:]
        w_next = Wc[i + 1] if i < 3 else None
        res = _tc_layer(a0, a1, deg0, deg1, h, bc[i].reshape(1, D),
                        lng[i].reshape(1, D), lnb[i].reshape(1, D), w_next)
        if i < 3:
            h, hs = res
        else:
            h = res[0]

    bt = jnp.full((NPAD,), G, jnp.int32).at[:N].set(batch.astype(jnp.int32))
    btf = jnp.broadcast_to(bt.astype(jnp.float32)[:, None], (NPAD, 8))
    Wn3p = jnp.zeros((D // 2, 8), jnp.float32).at[:, :2].set(Wn3)
    bn3p = jnp.zeros((1, 8), jnp.float32).at[:, :2].set(bn3.reshape(1, -1))
    nl8, gsum, gmax, gcnt = _tc_node_head(
        h, btf, Wn1, bn1.reshape(1, D), ng.reshape(1, D), nb.reshape(1, D),
        Wn2, bn2.reshape(1, -1), Wn3p, bn3p)

    Wg3p = jnp.zeros((D // 2, 8), jnp.float32).at[:, :2].set(Wg3)
    bg3p = jnp.zeros((1, 8), jnp.float32).at[:, :2].set(bg3.reshape(1, -1))
    gl8 = _tc_graph_head(
        gsum, gmax, gcnt, Wg1[:D], Wg1[D:], bg1.reshape(1, D),
        gg.reshape(1, D), gb.reshape(1, D), Wg2, bg2.reshape(1, -1),
        Wg3p, bg3p)

    return gl8[:, :2], nl8[:N, :2]
